# baseline (speedup 1.0000x reference)
import functools

import jax
import jax.numpy as jnp
from jax import lax
from jax.experimental import pallas as pl
from jax.experimental.pallas import tpu as pltpu
from jax.experimental.pallas import tpu_sc as plsc

_B = 4096
_N = 100000
_DIM = 64
_PD = 2 * _DIM
_VOLUME_TEMP = 1.0
_INTERSECTION_TEMP = 0.01
_SOFTPLUS_CONST = 2 * _INTERSECTION_TEMP * 0.5772156649015329

_info = plsc.get_sparse_core_info()
_NC, _NS, _NL = _info.num_cores, _info.num_subcores, _info.num_lanes
_NW = _NC * _NS
_FULL_COLS = (_N // 128) * 128
_NCT = _FULL_COLS // 128
_CHUNK = 256
_LISTCAP = 2 * _B
_OUTROWS = 2 * _B + _NL

_sc_mesh = plsc.VectorSubcoreMesh(core_axis_name="c", subcore_axis_name="s")


def _iota16():
    return lax.iota(jnp.int32, _NL)


@functools.partial(
    pl.kernel,
    mesh=_sc_mesh,
    compiler_params=pltpu.CompilerParams(needs_layout_passes=False),
    out_type=jax.ShapeDtypeStruct((_OUTROWS, _PD), jnp.float32),
    scratch_types=[
        pltpu.VMEM((_B,), jnp.int32),
        pltpu.VMEM((_B,), jnp.int32),
        pltpu.VMEM((_LISTCAP,), jnp.int32),
        pltpu.VMEM((_LISTCAP,), jnp.int32),
        pltpu.VMEM((_DIM, _CHUNK), jnp.float32),
        pltpu.VMEM((_DIM, _CHUNK), jnp.float32),
        pltpu.VMEM((_DIM, _CHUNK), jnp.float32),
        pltpu.VMEM((_DIM, _CHUNK), jnp.float32),
        pltpu.VMEM((32, _PD), jnp.float32),
        pltpu.VMEM((_NL, _PD), jnp.float32),
        pltpu.VMEM((_NL, _PD), jnp.float32),
        pltpu.VMEM((_NL,), jnp.int32),
        pltpu.VMEM((_NL,), jnp.int32),
        pltpu.SemaphoreType.DMA,
        pltpu.SemaphoreType.DMA,
        pltpu.SemaphoreType.DMA,
        pltpu.SemaphoreType.DMA,
    ],
)
def _sc_stream_gather(tt1, tt2, tailp, i1_hbm, i2_hbm, out,
                      idx1_v, idx2_v, list_v, hits_v, ch1a_v, ch2a_v,
                      ch1b_v, ch2b_v, tail_v, rows0_v, rows1_v,
                      bbuf0_v, bbuf1_v, sema, semb, sdma0, sdma1):
    wid = lax.axis_index("s") * _NC + lax.axis_index("c")
    start_ct = wid * 24 + jnp.minimum(wid, 13)
    n_ct = jnp.where(wid < 13, 25, 24)
    col_lo = start_ct * 128
    col_hi = (start_ct + n_ct) * 128
    scan_hi = jnp.where(wid == _NW - 1, _N, col_hi)
    nchunks = jnp.where(wid < 13, 13, 12)

    bufs = ((ch1a_v, ch2a_v, sema), (ch1b_v, ch2b_v, semb))
    for k in (0, 1):
        b1, b2, sk = bufs[k]
        ck = jnp.minimum(col_lo + k * _CHUNK, col_hi - _CHUNK)
        pltpu.async_copy(tt1.at[:, pl.ds(ck, _CHUNK)], b1, sk)
        pltpu.async_copy(tt2.at[:, pl.ds(ck, _CHUNK)], b2, sk)
    pltpu.sync_copy(i1_hbm, idx1_v)
    pltpu.sync_copy(i2_hbm, idx2_v)

    lane = _iota16()

    def scan_set(idx_ref, boff, cnt):
        def body(v, cnt):
            iv = idx_ref[pl.ds(v * _NL, _NL)]
            m = (iv >= col_lo) & (iv < scan_hi)
            m32 = m.astype(jnp.int32)
            pos = cnt + plsc.cumsum(m32) - m32
            packed = jnp.left_shift(iv, 13) + (v * _NL + lane + boff)
            plsc.store_scatter(list_v, [pos], packed, mask=m)
            return cnt + plsc.all_reduce_population_count(m)
        return lax.fori_loop(0, _B // _NL, body, cnt)

    cnt = scan_set(idx1_v, 0, jnp.zeros((_NL,), jnp.int32))
    cnt = scan_set(idx2_v, _B, cnt)
    n_list = jnp.max(cnt)
    n_lvregs = (n_list + _NL - 1) // _NL

    def process_chunk(c, ch1_v, ch2_v, sk):
        base = jnp.minimum(col_lo + c * _CHUNK, col_hi - _CHUNK)
        pltpu.make_async_copy(tt1.at[:, pl.ds(col_lo, _CHUNK)], ch1_v, sk).wait()
        pltpu.make_async_copy(tt2.at[:, pl.ds(col_lo, _CHUNK)], ch2_v, sk).wait()

        def scanb(g, hc):
            lv = list_v[pl.ds(g * _NL, _NL)]
            valid = (g * _NL + lane) < n_list
            iv = jnp.right_shift(lv, 13)
            m = valid & (iv >= base) & (iv < base + _CHUNK)
            m32 = m.astype(jnp.int32)
            pos = hc + plsc.cumsum(m32) - m32
            plsc.store_scatter(hits_v, [pos], lv, mask=m)
            return hc + plsc.all_reduce_population_count(m)
        nhits_v = lax.fori_loop(0, n_lvregs, scanb,
                                jnp.zeros((_NL,), jnp.int32))
        nhits = jnp.max(nhits_v)

        slots = ((rows0_v, bbuf0_v, sdma0), (rows1_v, bbuf1_v, sdma1))
        ngroups = (nhits + _NL - 1) // _NL

        def gpair(gp, _):
            for k in (0, 1):
                rows_v, bbuf_v, sd = slots[k]
                gg = gp * 2 + k
                @pl.when(gg < ngroups)
                def _():
                    @pl.when(gg >= 2)
                    def _():
                        pltpu.make_async_copy(rows_v, out.at[bbuf_v], sd).wait()
                    lv = hits_v[pl.ds(gg * _NL, _NL)]
                    valid = (gg * _NL + lane) < nhits
                    iv = jnp.right_shift(lv, 13)
                    bv = lv & (2 * _B - 1)
                    iloc = jnp.where(valid, iv - base, 0)
                    bbuf_v[...] = jnp.where(valid, bv, 2 * _B + lane)
                    for d in range(_DIM):
                        dsp = jnp.full((_NL,), d, jnp.int32)
                        v1 = plsc.load_gather(ch1_v, [dsp, iloc])
                        v2 = plsc.load_gather(ch2_v, [dsp, iloc])
                        plsc.store_scatter(rows_v, [lane, dsp], v1)
                        plsc.store_scatter(rows_v, [lane, dsp + _DIM], v2)
                    pltpu.async_copy(rows_v, out.at[bbuf_v], sd)
            return 0
        lax.fori_loop(0, (ngroups + 1) // 2, gpair, 0)
        for k in (0, 1):
            rows_v, bbuf_v, sd = slots[k]
            @pl.when(ngroups > k)
            def _():
                pltpu.make_async_copy(rows_v, out.at[bbuf_v], sd).wait()

    def chunk_loop(co, carry):
        for k in (0, 1):
            b1, b2, sk = bufs[k]
            c = co * 2 + k
            @pl.when(c < nchunks)
            def _():
                process_chunk(c, b1, b2, sk)
                nxt = jnp.minimum(col_lo + (c + 2) * _CHUNK,
                                  col_hi - _CHUNK)
                @pl.when(c + 2 < nchunks)
                def _():
                    pltpu.async_copy(tt1.at[:, pl.ds(nxt, _CHUNK)], b1, sk)
                    pltpu.async_copy(tt2.at[:, pl.ds(nxt, _CHUNK)], b2, sk)
        return carry

    lax.fori_loop(0, 7, chunk_loop, 0)


    @pl.when(wid == _NW - 1)
    def _tail():
        pltpu.sync_copy(tailp, tail_v)

        def tgroup(gg, _):
            def tscan(g, hc):
                lv = list_v[pl.ds(g * _NL, _NL)]
                valid = (g * _NL + lane) < n_list
                iv = jnp.right_shift(lv, 13)
                m = valid & (iv >= _FULL_COLS)
                m32 = m.astype(jnp.int32)
                pos = hc + plsc.cumsum(m32) - m32
                plsc.store_scatter(hits_v, [pos], lv, mask=m)
                return hc + plsc.all_reduce_population_count(m)
            return tscan(gg, _)
        nhits_v = lax.fori_loop(0, n_lvregs, tgroup,
                                jnp.zeros((_NL,), jnp.int32))
        nhits = jnp.max(nhits_v)

        def group(gg, _):
            lv = hits_v[pl.ds(gg * _NL, _NL)]
            valid = (gg * _NL + lane) < nhits
            iv = jnp.right_shift(lv, 13)
            bv = lv & (2 * _B - 1)
            iloc = jnp.where(valid, iv - _FULL_COLS, 0)
            bbuf0_v[...] = jnp.where(valid, bv, 2 * _B + lane)
            for d in range(_PD):
                dsp = jnp.full((_NL,), d, jnp.int32)
                v = plsc.load_gather(tail_v, [iloc, dsp])
                plsc.store_scatter(rows0_v, [lane, dsp], v)
            pltpu.async_copy(rows0_v, out.at[bbuf0_v], sdma0).wait()
            return 0
        lax.fori_loop(0, (nhits + _NL - 1) // _NL, group, 0)


def _softplus(x):
    return jnp.logaddexp(x, 0.0)


def _box_math_body(b1_ref, b2_ref, out_ref):
    c1 = b1_ref[:, :_DIM]
    w1 = _softplus(b1_ref[:, _DIM:]) * 0.5
    c2 = b2_ref[:, :_DIM]
    w2 = _softplus(b2_ref[:, _DIM:]) * 0.5
    min1 = c1 - w1
    max1 = c1 + w1
    min2 = c2 - w2
    max2 = c2 + w2
    t = _INTERSECTION_TEMP
    meet_min = t * jnp.logaddexp(min1 / t, min2 / t)
    meet_max = -t * jnp.logaddexp(-max1 / t, -max2 / t)
    meet_min = jnp.maximum(meet_min, jnp.maximum(min1, min2))
    meet_max = jnp.minimum(meet_max, jnp.minimum(max1, max2))
    log_overlap = jnp.sum(
        jnp.log(_VOLUME_TEMP * _softplus(
            (meet_max - meet_min - _SOFTPLUS_CONST) / _VOLUME_TEMP) + 1e-20),
        axis=-1)
    log_rhs = jnp.sum(
        jnp.log(_VOLUME_TEMP * _softplus(
            (max2 - min2 - _SOFTPLUS_CONST) / _VOLUME_TEMP) + 1e-20),
        axis=-1)
    out_ref[...] = log_overlap - log_rhs


_TC_BLOCK = 512


def _tc_math(rows):
    grid = _B // _TC_BLOCK
    return pl.pallas_call(
        _box_math_body,
        grid=(grid,),
        in_specs=[pl.BlockSpec((_TC_BLOCK, _PD), lambda i: (i, 0)),
                  pl.BlockSpec((_TC_BLOCK, _PD), lambda i: (i + 8, 0))],
        out_specs=pl.BlockSpec((_TC_BLOCK,), lambda i: (i,)),
        out_shape=jax.ShapeDtypeStruct((_B,), jnp.float32),
    )(rows, rows)


def kernel(idx1, idx2, emb1, emb2, embs1_weight, embs2_weight):
    del emb1, emb2
    i1 = idx1.astype(jnp.int32)
    i2 = idx2.astype(jnp.int32)
    tailp = jnp.concatenate(
        [embs1_weight[_FULL_COLS:], embs2_weight[_FULL_COLS:]], axis=1)
    rows = _sc_stream_gather(embs1_weight.T, embs2_weight.T, tailp, i1, i2)
    return _tc_math(rows)


# speedup vs baseline: 1.2186x; 1.2186x over previous
import functools

import jax
import jax.numpy as jnp
from jax import lax
from jax.experimental import pallas as pl
from jax.experimental.pallas import tpu as pltpu
from jax.experimental.pallas import tpu_sc as plsc

_B = 4096
_N = 100000
_DIM = 64
_PD = 2 * _DIM
_VOLUME_TEMP = 1.0
_INTERSECTION_TEMP = 0.01
_SOFTPLUS_CONST = 2 * _INTERSECTION_TEMP * 0.5772156649015329

_info = plsc.get_sparse_core_info()
_NC, _NS, _NL = _info.num_cores, _info.num_subcores, _info.num_lanes
_NW = _NC * _NS
_BPW = _B // _NW


_TW = 16384


def _transpose_body(tt1_ref, tt2_ref, o_ref):
    row = lax.broadcasted_iota(jnp.int32, (_DIM, _PD), 0)
    col = lax.broadcasted_iota(jnp.int32, (_DIM, _PD), 1)
    i_low = (row == col).astype(jnp.float32)
    i_high = (row + _DIM == col).astype(jnp.float32)
    dn = (((0,), (0,)), ((), ()))
    o_ref[...] = (
        lax.dot_general(tt1_ref[...], i_low, dn,
                        preferred_element_type=jnp.float32)
        + lax.dot_general(tt2_ref[...], i_high, dn,
                          preferred_element_type=jnp.float32))


def _tc_transpose(tt1, tt2):
    grid = (_N + _TW - 1) // _TW
    in_spec = pl.BlockSpec((_DIM, _TW), lambda i: (0, i))
    return pl.pallas_call(
        _transpose_body,
        grid=(grid,),
        in_specs=[in_spec, in_spec],
        out_specs=pl.BlockSpec((_TW, _PD), lambda i: (i, 0)),
        out_shape=jax.ShapeDtypeStruct((_N, _PD), jnp.float32),
    )(tt1, tt2)


_sc_mesh = plsc.VectorSubcoreMesh(core_axis_name="c", subcore_axis_name="s")


@functools.partial(
    pl.kernel,
    mesh=_sc_mesh,
    out_type=[jax.ShapeDtypeStruct((_B, _PD), jnp.float32)] * 2,
    scratch_types=[
        pltpu.VMEM((_BPW,), jnp.int32),
        pltpu.VMEM((_BPW,), jnp.int32),
        pltpu.VMEM((_BPW, _PD), jnp.float32),
        pltpu.VMEM((_BPW, _PD), jnp.float32),
        pltpu.SemaphoreType.DMA,
    ],
)
def _sc_gather(t_hbm, i1_hbm, i2_hbm,
               o1, o2,
               idx1_v, idx2_v, r1_v, r2_v, sem):
    wid = lax.axis_index("s") * _NC + lax.axis_index("c")
    base = wid * _BPW
    pltpu.sync_copy(i1_hbm.at[pl.ds(base, _BPW)], idx1_v)
    pltpu.sync_copy(i2_hbm.at[pl.ds(base, _BPW)], idx2_v)
    d1 = pltpu.async_copy(t_hbm.at[idx1_v], r1_v, sem)
    d2 = pltpu.async_copy(t_hbm.at[idx2_v], r2_v, sem)
    d1.wait()
    d2.wait()
    pltpu.sync_copy(r1_v, o1.at[pl.ds(base, _BPW)])
    pltpu.sync_copy(r2_v, o2.at[pl.ds(base, _BPW)])


def _softplus(x):
    return jnp.logaddexp(x, 0.0)


def _box_math_body(b1_ref, b2_ref, out_ref):
    c1 = b1_ref[:, :_DIM]
    w1 = _softplus(b1_ref[:, _DIM:]) * 0.5
    c2 = b2_ref[:, :_DIM]
    w2 = _softplus(b2_ref[:, _DIM:]) * 0.5
    min1 = c1 - w1
    max1 = c1 + w1
    min2 = c2 - w2
    max2 = c2 + w2
    t = _INTERSECTION_TEMP
    meet_min = t * jnp.logaddexp(min1 / t, min2 / t)
    meet_max = -t * jnp.logaddexp(-max1 / t, -max2 / t)
    meet_min = jnp.maximum(meet_min, jnp.maximum(min1, min2))
    meet_max = jnp.minimum(meet_max, jnp.minimum(max1, max2))
    log_overlap = jnp.sum(
        jnp.log(_VOLUME_TEMP * _softplus(
            (meet_max - meet_min - _SOFTPLUS_CONST) / _VOLUME_TEMP) + 1e-20),
        axis=-1)
    log_rhs = jnp.sum(
        jnp.log(_VOLUME_TEMP * _softplus(
            (max2 - min2 - _SOFTPLUS_CONST) / _VOLUME_TEMP) + 1e-20),
        axis=-1)
    out_ref[...] = log_overlap - log_rhs


_TC_BLOCK = 512


def _tc_math(r1, r2):
    grid = _B // _TC_BLOCK
    in_spec = pl.BlockSpec((_TC_BLOCK, _PD), lambda i: (i, 0))
    return pl.pallas_call(
        _box_math_body,
        grid=(grid,),
        in_specs=[in_spec, in_spec],
        out_specs=pl.BlockSpec((_TC_BLOCK,), lambda i: (i,)),
        out_shape=jax.ShapeDtypeStruct((_B,), jnp.float32),
    )(r1, r2)


def kernel(idx1, idx2, emb1, emb2, embs1_weight, embs2_weight):
    del emb1, emb2
    i1 = idx1.astype(jnp.int32)
    i2 = idx2.astype(jnp.int32)
    tp = _tc_transpose(embs1_weight.T, embs2_weight.T)
    r1, r2 = _sc_gather(tp, i1, i2)
    return _tc_math(r1, r2)
